# baseline (speedup 1.0000x reference)
import functools

import jax
import jax.numpy as jnp
from jax import lax
from jax.experimental import pallas as pl
from jax.experimental.pallas import tpu as pltpu
from jax.experimental.pallas import tpu_sc as plsc

N_SPEC = 1024
B = 1024
R1_N = 4096
R2_N = 16384
RTOT = R1_N + R2_N
NC = 2
NS = 16
NW = NC * NS
BPW = B // NW
L = 16
YW = (N_SPEC + 1) * BPW
AW = N_SPEC * BPW

RBLK = 512
RBUF = RBLK * BPW
NBLK = RTOT // RBLK


_WQ = 4
_RB = 512


def _rates_tc(a_ref, e_ref, t_ref, o_ref):
    r = a_ref[...] * jnp.exp(-e_ref[...] / t_ref[...])
    o_ref[...] = r.reshape(_WQ, BPW, _RB).transpose(0, 2, 1)


_rates_call = pl.pallas_call(
    _rates_tc,
    grid=(NW // _WQ, RTOT // _RB),
    in_specs=[
        pl.BlockSpec((1, _RB), lambda i, j: (0, j)),
        pl.BlockSpec((1, _RB), lambda i, j: (0, j)),
        pl.BlockSpec((_WQ * BPW, 1), lambda i, j: (i, 0)),
    ],
    out_specs=pl.BlockSpec((_WQ, _RB, BPW), lambda i, j: (i, j, 0)),
    out_shape=jax.ShapeDtypeStruct((NW, RTOT, BPW), jnp.float32),
)


def _build_sc_kernel():
    mesh = plsc.VectorSubcoreMesh(core_axis_name="c", subcore_axis_name="s")

    @functools.partial(
        pl.kernel,
        mesh=mesh,
        out_type=jax.ShapeDtypeStruct((NW, AW), jnp.float32),
        scratch_types=[
            pltpu.VMEM((YW,), jnp.float32),
            pltpu.VMEM((AW,), jnp.float32),
            pltpu.VMEM((RBUF,), jnp.float32),
            pltpu.VMEM((RBUF,), jnp.float32),
            pltpu.VMEM((RBLK,), jnp.int32),
            pltpu.VMEM((RBLK,), jnp.int32),
            pltpu.VMEM((RBLK,), jnp.int32),
            pltpu.VMEM((RBLK,), jnp.int32),
            pltpu.VMEM((RBLK,), jnp.int32),
            pltpu.VMEM((RBLK,), jnp.int32),
            pltpu.SemaphoreType.DMA,
            pltpu.SemaphoreType.DMA,
        ],
    )
    def reaction_kernel(yr_hbm, rates_hbm, i_hbm, j_hbm, p_hbm, out_hbm,
                        y_v, acc_v, r0_v, r1_v, i0_v, i1_v, j0_v, j1_v,
                        p0_v, p1_v, sem0, sem1):
        wid = lax.axis_index("s") * NC + lax.axis_index("c")

        pltpu.sync_copy(yr_hbm.at[wid], y_v)

        def zero_body(s, carry):
            acc_v[pl.ds(s * L, L)] = jnp.zeros((L,), jnp.float32)
            return carry
        lax.fori_loop(0, AW // L, zero_body, 0)

        bufs = ((r0_v, i0_v, j0_v, p0_v, sem0),
                (r1_v, i1_v, j1_v, p1_v, sem1))

        def copies(buf, blk):
            r_v, i_v, j_v, p_v, sem = buf
            base = blk * RBLK
            return (
                (rates_hbm.at[wid, pl.ds(base * BPW, RBUF)], r_v, sem),
                (i_hbm.at[pl.ds(base, RBLK)], i_v, sem),
                (j_hbm.at[pl.ds(base, RBLK)], j_v, sem),
                (p_hbm.at[pl.ds(base, RBLK)], p_v, sem),
            )

        def issue(buf, blk):
            for src, dst, sem in copies(buf, blk):
                pltpu.async_copy(src, dst, sem)

        def drain(buf, blk):
            for src, dst, sem in copies(buf, blk):
                pltpu.make_async_copy(src, dst, sem).wait()

        def compute(buf):
            r_v, i_v, j_v, p_v, sem = buf

            def group(g, carry):
                gb = g * L
                iv16 = i_v[pl.ds(gb, L)]
                jv16 = j_v[pl.ds(gb, L)]
                pv16 = p_v[pl.ds(gb, L)]
                rb = gb * BPW
                for k in range(L):
                    i = iv16[k]
                    j = jv16[k]
                    p = pv16[k]
                    ro = rb + k * BPW
                    rg0 = r_v[pl.ds(ro, L)]
                    rg1 = r_v[pl.ds(ro + L, L)]
                    yi0 = y_v[pl.ds(i, L)]
                    yj0 = y_v[pl.ds(j, L)]
                    plsc.addupdate(acc_v.at[pl.ds(p, L)], yi0 * yj0 * rg0)
                    yi1 = y_v[pl.ds(i + L, L)]
                    yj1 = y_v[pl.ds(j + L, L)]
                    plsc.addupdate(acc_v.at[pl.ds(p + L, L)], yi1 * yj1 * rg1)
                return carry
            lax.fori_loop(0, RBLK // L, group, 0)

        issue(bufs[0], 0)
        issue(bufs[1], 1)

        def block_pair(b, carry):
            blk0 = b * 2
            drain(bufs[0], blk0)
            compute(bufs[0])

            @pl.when(blk0 + 2 < NBLK)
            def _():
                issue(bufs[0], blk0 + 2)

            drain(bufs[1], blk0 + 1)
            compute(bufs[1])

            @pl.when(blk0 + 3 < NBLK)
            def _():
                issue(bufs[1], blk0 + 3)
            return carry
        lax.fori_loop(0, NBLK // 2, block_pair, 0)

        pltpu.sync_copy(acc_v, out_hbm.at[wid])

    return reaction_kernel


_SC_KERNEL = _build_sc_kernel()


def kernel(t_in, y_in, inds_1r, inds_1p, inds_2r, inds_2p, A1, E1, A2, E2):
    iv = jnp.concatenate([inds_1r, inds_2r[:, 0]]) * BPW
    jv = jnp.concatenate([jnp.full((R1_N,), N_SPEC, jnp.int32),
                          inds_2r[:, 1]]) * BPW
    pv = jnp.concatenate([inds_1p, inds_2p]) * BPW
    av = jnp.concatenate([A1, A2]).reshape(1, RTOT)
    ev = jnp.concatenate([E1, E2]).reshape(1, RTOT)

    rates = _rates_call(av, ev, t_in).reshape(NW, RTOT * BPW)

    y_aug = jnp.concatenate([y_in, jnp.ones((B, 1), jnp.float32)], axis=1)
    yr = y_aug.reshape(NW, BPW, N_SPEC + 1).transpose(0, 2, 1).reshape(NW, YW)

    out = _SC_KERNEL(yr, rates, iv, jv, pv)
    return out.reshape(NW, N_SPEC, BPW).transpose(0, 2, 1).reshape(B, N_SPEC)


# speedup vs baseline: 2.3520x; 2.3520x over previous
import functools

import jax
import jax.numpy as jnp
from jax import lax
from jax.experimental import pallas as pl
from jax.experimental.pallas import tpu as pltpu
from jax.experimental.pallas import tpu_sc as plsc

N_SPEC = 1024
B = 1024
R1_N = 4096
R2_N = 16384
RBLK = 4096
NBLK2 = R2_N // RBLK
NC = 2
NS = 16
NW = NC * NS
BPW = B // NW
L = 16
YW = N_SPEC * BPW
AW = N_SPEC * BPW


def _build_sc_kernel():
    mesh = plsc.VectorSubcoreMesh(core_axis_name="c", subcore_axis_name="s")

    @functools.partial(
        pl.kernel,
        mesh=mesh,
        out_type=jax.ShapeDtypeStruct((NW, AW), jnp.float32),
        scratch_types=[
            pltpu.VMEM((YW,), jnp.float32),
            pltpu.VMEM((AW,), jnp.float32),
            pltpu.VMEM((RBLK,), jnp.int32),
            pltpu.VMEM((RBLK,), jnp.int32),
            pltpu.VMEM((RBLK,), jnp.int32),
            pltpu.VMEM((RBLK,), jnp.float32),
            pltpu.VMEM((RBLK,), jnp.float32),
            pltpu.VMEM((BPW,), jnp.float32),
        ],
    )
    def reaction_kernel(yr_hbm, t_hbm, i1_hbm, p1_hbm, a1_hbm, e1_hbm,
                        i2_hbm, j2_hbm, p2_hbm, a2_hbm, e2_hbm,
                        out_hbm, y_v, acc_v, i_v, j_v, p_v, a_v, e_v, t_v):
        wid = lax.axis_index("s") * NC + lax.axis_index("c")

        pltpu.sync_copy(yr_hbm.at[wid], y_v)
        pltpu.sync_copy(t_hbm.at[pl.ds(wid * BPW, BPW)], t_v)

        def zero_body(s, carry):
            acc_v[pl.ds(s * L, L)] = jnp.zeros((L,), jnp.float32)
            return carry
        lax.fori_loop(0, AW // L, zero_body, 0)

        invt0 = 1.0 / t_v[pl.ds(0, L)]
        invt1 = 1.0 / t_v[pl.ds(L, L)]

        pltpu.sync_copy(i1_hbm, i_v)
        pltpu.sync_copy(p1_hbm, p_v)
        pltpu.sync_copy(a1_hbm, a_v)
        pltpu.sync_copy(e1_hbm, e_v)

        def body1(g, carry):
            it0, it1 = carry
            gb = g * L
            iv16 = i_v[pl.ds(gb, L)]
            pv16 = p_v[pl.ds(gb, L)]
            av16 = a_v[pl.ds(gb, L)]
            ev16 = e_v[pl.ds(gb, L)]
            for k in range(L):
                i = iv16[k]
                p = pv16[k]
                a = av16[k]
                ep = ev16[k]
                yi0 = y_v[pl.ds(i, L)]
                term0 = yi0 * (a * jnp.exp(ep * it0))
                plsc.addupdate(acc_v.at[pl.ds(p, L)], term0)
                yi1 = y_v[pl.ds(i + L, L)]
                term1 = yi1 * (a * jnp.exp(ep * it1))
                plsc.addupdate(acc_v.at[pl.ds(p + L, L)], term1)
            return carry
        lax.fori_loop(0, R1_N // L, body1, (invt0, invt1))

        for blk in range(NBLK2):
            base = blk * RBLK
            pltpu.sync_copy(i2_hbm.at[pl.ds(base, RBLK)], i_v)
            pltpu.sync_copy(j2_hbm.at[pl.ds(base, RBLK)], j_v)
            pltpu.sync_copy(p2_hbm.at[pl.ds(base, RBLK)], p_v)
            pltpu.sync_copy(a2_hbm.at[pl.ds(base, RBLK)], a_v)
            pltpu.sync_copy(e2_hbm.at[pl.ds(base, RBLK)], e_v)

            def body2(g, carry):
                it0, it1 = carry
                gb = g * L
                iv16 = i_v[pl.ds(gb, L)]
                jv16 = j_v[pl.ds(gb, L)]
                pv16 = p_v[pl.ds(gb, L)]
                av16 = a_v[pl.ds(gb, L)]
                ev16 = e_v[pl.ds(gb, L)]
                for k in range(L):
                    i = iv16[k]
                    j = jv16[k]
                    p = pv16[k]
                    a = av16[k]
                    ep = ev16[k]
                    yi0 = y_v[pl.ds(i, L)]
                    yj0 = y_v[pl.ds(j, L)]
                    term0 = (yi0 * yj0) * (a * jnp.exp(ep * it0))
                    plsc.addupdate(acc_v.at[pl.ds(p, L)], term0)
                    yi1 = y_v[pl.ds(i + L, L)]
                    yj1 = y_v[pl.ds(j + L, L)]
                    term1 = (yi1 * yj1) * (a * jnp.exp(ep * it1))
                    plsc.addupdate(acc_v.at[pl.ds(p + L, L)], term1)
                return carry
            lax.fori_loop(0, RBLK // L, body2, (invt0, invt1))

        pltpu.sync_copy(acc_v, out_hbm.at[wid])

    return reaction_kernel


_SC_KERNEL = _build_sc_kernel()


def kernel(t_in, y_in, inds_1r, inds_1p, inds_2r, inds_2p, A1, E1, A2, E2):
    i1 = inds_1r * BPW
    p1 = inds_1p * BPW
    ep1 = -E1
    i2 = inds_2r[:, 0] * BPW
    j2 = inds_2r[:, 1] * BPW
    p2 = inds_2p * BPW
    ep2 = -E2
    yr = y_in.reshape(NW, BPW, N_SPEC).transpose(0, 2, 1).reshape(NW, YW)
    tflat = t_in.reshape(B)

    out = _SC_KERNEL(yr, tflat, i1, p1, A1, ep1, i2, j2, p2, A2, ep2)
    return out.reshape(NW, N_SPEC, BPW).transpose(0, 2, 1).reshape(B, N_SPEC)


# speedup vs baseline: 2.3935x; 1.0176x over previous
import functools

import jax
import jax.numpy as jnp
from jax import lax
from jax.experimental import pallas as pl
from jax.experimental.pallas import tpu as pltpu
from jax.experimental.pallas import tpu_sc as plsc

N_SPEC = 1024
B = 1024
R1_N = 4096
R2_N = 16384
RBLK = 4096
NBLK2 = R2_N // RBLK
NC = 2
NS = 16
NW = NC * NS
BPW = B // NW
L = 16
YW = N_SPEC * BPW
AW = N_SPEC * BPW


def _build_sc_kernel():
    mesh = plsc.VectorSubcoreMesh(core_axis_name="c", subcore_axis_name="s")

    @functools.partial(
        pl.kernel,
        mesh=mesh,
        out_type=jax.ShapeDtypeStruct((NW, AW), jnp.float32),
        scratch_types=[
            pltpu.VMEM((YW,), jnp.float32),
            pltpu.VMEM((AW,), jnp.float32),
            pltpu.VMEM((RBLK,), jnp.int32),
            pltpu.VMEM((RBLK,), jnp.float32),
            pltpu.VMEM((RBLK,), jnp.float32),
            pltpu.VMEM((BPW,), jnp.float32),
        ],
    )
    def reaction_kernel(yr_hbm, t_hbm, pk1_hbm, a1_hbm, e1_hbm,
                        pk2_hbm, a2_hbm, e2_hbm,
                        out_hbm, y_v, acc_v, pk_v, a_v, e_v, t_v):
        wid = lax.axis_index("s") * NC + lax.axis_index("c")

        pltpu.sync_copy(yr_hbm.at[wid], y_v)
        pltpu.sync_copy(t_hbm.at[pl.ds(wid * BPW, BPW)], t_v)

        def zero_body(s, carry):
            acc_v[pl.ds(s * L, L)] = jnp.zeros((L,), jnp.float32)
            return carry
        lax.fori_loop(0, AW // L, zero_body, 0)

        invt0 = 1.0 / t_v[pl.ds(0, L)]
        invt1 = 1.0 / t_v[pl.ds(L, L)]

        pltpu.sync_copy(pk1_hbm, pk_v)
        pltpu.sync_copy(a1_hbm, a_v)
        pltpu.sync_copy(e1_hbm, e_v)

        def body1(g, carry):
            it0, it1 = carry
            gb = g * L
            pkv16 = pk_v[pl.ds(gb, L)]
            av16 = a_v[pl.ds(gb, L)]
            ev16 = e_v[pl.ds(gb, L)]
            for k in range(L):
                pk = pkv16[k]
                i = (pk & 1023) << 5
                p = (pk >> 10) << 5
                a = av16[k]
                ep = ev16[k]
                yi0 = y_v[pl.ds(i, L)]
                term0 = yi0 * (a * jnp.exp(ep * it0))
                plsc.addupdate(acc_v.at[pl.ds(p, L)], term0)
                yi1 = y_v[pl.ds(i + L, L)]
                term1 = yi1 * (a * jnp.exp(ep * it1))
                plsc.addupdate(acc_v.at[pl.ds(p + L, L)], term1)
            return carry
        lax.fori_loop(0, R1_N // L, body1, (invt0, invt1))

        for blk in range(NBLK2):
            base = blk * RBLK
            pltpu.sync_copy(pk2_hbm.at[pl.ds(base, RBLK)], pk_v)
            pltpu.sync_copy(a2_hbm.at[pl.ds(base, RBLK)], a_v)
            pltpu.sync_copy(e2_hbm.at[pl.ds(base, RBLK)], e_v)

            def body2(g, carry):
                it0, it1 = carry
                gb = g * L
                pkv16 = pk_v[pl.ds(gb, L)]
                av16 = a_v[pl.ds(gb, L)]
                ev16 = e_v[pl.ds(gb, L)]
                for k in range(L):
                    pk = pkv16[k]
                    i = (pk & 1023) << 5
                    j = ((pk >> 10) & 1023) << 5
                    p = (pk >> 20) << 5
                    a = av16[k]
                    ep = ev16[k]
                    yi0 = y_v[pl.ds(i, L)]
                    yj0 = y_v[pl.ds(j, L)]
                    term0 = (yi0 * yj0) * (a * jnp.exp(ep * it0))
                    plsc.addupdate(acc_v.at[pl.ds(p, L)], term0)
                    yi1 = y_v[pl.ds(i + L, L)]
                    yj1 = y_v[pl.ds(j + L, L)]
                    term1 = (yi1 * yj1) * (a * jnp.exp(ep * it1))
                    plsc.addupdate(acc_v.at[pl.ds(p + L, L)], term1)
                return carry
            lax.fori_loop(0, RBLK // L, body2, (invt0, invt1))

        pltpu.sync_copy(acc_v, out_hbm.at[wid])

    return reaction_kernel


_SC_KERNEL = _build_sc_kernel()


def kernel(t_in, y_in, inds_1r, inds_1p, inds_2r, inds_2p, A1, E1, A2, E2):
    pk1 = inds_1r.astype(jnp.int32) | (inds_1p.astype(jnp.int32) << 10)
    ep1 = -E1
    pk2 = (inds_2r[:, 0].astype(jnp.int32)
           | (inds_2r[:, 1].astype(jnp.int32) << 10)
           | (inds_2p.astype(jnp.int32) << 20))
    ep2 = -E2
    yr = y_in.reshape(NW, BPW, N_SPEC).transpose(0, 2, 1).reshape(NW, YW)
    tflat = t_in.reshape(B)

    out = _SC_KERNEL(yr, tflat, pk1, A1, ep1, pk2, A2, ep2)
    return out.reshape(NW, N_SPEC, BPW).transpose(0, 2, 1).reshape(B, N_SPEC)
